# Initial kernel scaffold; baseline (speedup 1.0000x reference)
import jax
import jax.numpy as jnp
from jax.experimental import pallas as pl


def kernel(x, edge_index, W0, b0, g0, be0, W1, b1, g1, be1, W2, b2, g2, be2):
    raise NotImplementedError("write your pallas kernel here")


# speedup vs baseline: 12.6442x; 12.6442x over previous
import functools
import jax
import jax.numpy as jnp
from jax import lax
from jax.experimental import pallas as pl
from jax.experimental.pallas import tpu as pltpu
from jax.experimental.pallas import tpu_sc as plsc

NC = 2
NS = 16
NW = NC * NS
CHUNK = 128
RB = 1000
RC = 200


def _mesh():
    return plsc.VectorSubcoreMesh(core_axis_name="c", subcore_axis_name="s")


@functools.lru_cache(maxsize=None)
def _make_deg_kernel(N, EC):
    base, rem = EC // NW, EC % NW
    nrc = N // RC
    rc_base, rc_rem = nrc // NS, nrc % NS

    @functools.partial(
        pl.kernel,
        out_type=jax.ShapeDtypeStruct((NC, N, 8), jnp.float32),
        mesh=_mesh(),
        scratch_types=[
            pltpu.VMEM((1, CHUNK), jnp.int32),
            pltpu.VMEM((CHUNK, 8), jnp.float32),
            pltpu.VMEM((RC, 8), jnp.float32),
            pltpu.VMEM_SHARED((N, 8), jnp.float32),
        ],
    )
    def deg_kernel(dst_hbm, ones_hbm, zeros_hbm, out_hbm,
                   idx_v, ones_v, buf_v, acc_sh):
        c = lax.axis_index("c")
        s = lax.axis_index("s")
        w = s * NC + c
        pltpu.sync_copy(zeros_hbm, buf_v)
        pltpu.sync_copy(ones_hbm, ones_v)
        nrc_mine = rc_base + jnp.where(s < rc_rem, 1, 0)

        def zbody(k, carry):
            pltpu.sync_copy(buf_v, acc_sh.at[pl.ds((s + NS * k) * RC, RC)])
            return carry

        lax.fori_loop(0, nrc_mine, zbody, 0)
        plsc.subcore_barrier()
        start = base * w + jnp.minimum(w, rem)
        nch = base + jnp.where(w < rem, 1, 0)

        def body(j, carry):
            pltpu.sync_copy(dst_hbm.at[start + j], idx_v)
            pltpu.sync_copy(ones_v, acc_sh.at[idx_v.at[0]], add=True)
            return carry

        lax.fori_loop(0, nch, body, 0)
        plsc.subcore_barrier()

        def obody(k, carry):
            t = (s + NS * k) * RC
            pltpu.sync_copy(acc_sh.at[pl.ds(t, RC)], buf_v)
            pltpu.sync_copy(buf_v, out_hbm.at[c, pl.ds(t, RC)])
            return carry

        lax.fori_loop(0, nrc_mine, obody, 0)

    return deg_kernel


@functools.lru_cache(maxsize=None)
def _make_edge_kernel(N, D, EC):
    base, rem = EC // NW, EC % NW
    nrc = N // RC
    rc_base, rc_rem = nrc // NS, nrc % NS

    @functools.partial(
        pl.kernel,
        out_type=jax.ShapeDtypeStruct((NC, N, D), jnp.float32),
        mesh=_mesh(),
        scratch_types=[
            pltpu.VMEM((1, CHUNK), jnp.int32),
            pltpu.VMEM((1, CHUNK), jnp.int32),
            pltpu.VMEM((CHUNK, D), jnp.float32),
            pltpu.VMEM((RC, D), jnp.float32),
            pltpu.VMEM_SHARED((N, D), jnp.float32),
            pltpu.SemaphoreType.DMA,
        ],
    )
    def edge_kernel(hwp_hbm, src_hbm, dst_hbm, zeros_hbm, out_hbm,
                    src_v, dst_v, rows_v, buf_v, acc_sh, sem):
        c = lax.axis_index("c")
        s = lax.axis_index("s")
        w = s * NC + c
        pltpu.sync_copy(zeros_hbm, buf_v)
        nrc_mine = rc_base + jnp.where(s < rc_rem, 1, 0)

        def zbody(k, carry):
            pltpu.sync_copy(buf_v, acc_sh.at[pl.ds((s + NS * k) * RC, RC)])
            return carry

        lax.fori_loop(0, nrc_mine, zbody, 0)
        plsc.subcore_barrier()
        start = base * w + jnp.minimum(w, rem)
        nch = base + jnp.where(w < rem, 1, 0)

        def body(j, carry):
            ch = start + j
            pltpu.sync_copy(src_hbm.at[ch], src_v)
            pltpu.sync_copy(dst_hbm.at[ch], dst_v)
            pltpu.async_copy(hwp_hbm.at[src_v.at[0]], rows_v, sem).wait()
            pltpu.sync_copy(rows_v, acc_sh.at[dst_v.at[0]], add=True)
            return carry

        lax.fori_loop(0, nch, body, 0)
        plsc.subcore_barrier()

        def obody(k, carry):
            t = (s + NS * k) * RC
            pltpu.sync_copy(acc_sh.at[pl.ds(t, RC)], buf_v)
            pltpu.sync_copy(buf_v, out_hbm.at[c, pl.ds(t, RC)])
            return carry

        lax.fori_loop(0, nrc_mine, obody, 0)

    return edge_kernel


def _dis_body(degp_ref, dis_ref):
    d = degp_ref[0, :, 0:1] + degp_ref[1, :, 0:1] + 1.0
    dis_ref[...] = lax.rsqrt(d)


def _t1_body(h_ref, w_ref, dis_ref, hwp_ref):
    hwp_ref[...] = (
        jnp.dot(h_ref[...], w_ref[...], preferred_element_type=jnp.float32)
        * dis_ref[...]
    )


def _post(h, hwp, accp0, accp1, dis, b, g, be):
    out = (accp0 + accp1 + hwp) * dis + b
    mu = jnp.mean(out, axis=-1, keepdims=True)
    xm = out - mu
    var = jnp.mean(xm * xm, axis=-1, keepdims=True)
    out = xm * lax.rsqrt(var + 1e-5) * g + be
    return h + jnp.maximum(out, 0.0)


def _t2_body(h_ref, hwp_ref, accp_ref, dis_ref, b_ref, g_ref, be_ref, wn_ref,
             hn_ref, hwpn_ref):
    hn = _post(h_ref[...], hwp_ref[...], accp_ref[0], accp_ref[1],
               dis_ref[...], b_ref[...], g_ref[...], be_ref[...])
    hn_ref[...] = hn
    hwpn_ref[...] = (
        jnp.dot(hn, wn_ref[...], preferred_element_type=jnp.float32)
        * dis_ref[...]
    )


def _t3_body(h_ref, hwp_ref, accp_ref, dis_ref, b_ref, g_ref, be_ref, hn_ref):
    hn_ref[...] = _post(h_ref[...], hwp_ref[...], accp_ref[0], accp_ref[1],
                        dis_ref[...], b_ref[...], g_ref[...], be_ref[...])


def _row_spec(D):
    return pl.BlockSpec((RB, D), lambda i: (i, 0))


def kernel(x, edge_index, W0, b0, g0, be0, W1, b1, g1, be1, W2, b2, g2, be2):
    N, D = x.shape
    E = edge_index.shape[1]
    EC = E // CHUNK
    src3d = edge_index[0].reshape(EC, 1, CHUNK)
    dst3d = edge_index[1].reshape(EC, 1, CHUNK)
    ones8 = jnp.ones((CHUNK, 8), jnp.float32)
    zdeg = jnp.zeros((RC, 8), jnp.float32)
    zrow = jnp.zeros((RC, D), jnp.float32)
    grid = (N // RB,)

    degp = _make_deg_kernel(N, EC)(dst3d, ones8, zdeg)

    dis = pl.pallas_call(
        _dis_body,
        grid=grid,
        in_specs=[pl.BlockSpec((NC, RB, 8), lambda i: (0, i, 0))],
        out_specs=pl.BlockSpec((RB, 1), lambda i: (i, 0)),
        out_shape=jax.ShapeDtypeStruct((N, 1), jnp.float32),
    )(degp)

    full = lambda a, b: pl.BlockSpec((a, b), lambda i: (0, 0))
    dis_spec = pl.BlockSpec((RB, 1), lambda i: (i, 0))
    accp_spec = pl.BlockSpec((NC, RB, D), lambda i: (0, i, 0))

    hwp = pl.pallas_call(
        _t1_body,
        grid=grid,
        in_specs=[_row_spec(D), full(D, D), dis_spec],
        out_specs=_row_spec(D),
        out_shape=jax.ShapeDtypeStruct((N, D), jnp.float32),
    )(x, W0, dis)

    edge_k = _make_edge_kernel(N, D, EC)
    h = x
    layer_params = [(b0, g0, be0, W1), (b1, g1, be1, W2), (b2, g2, be2, None)]
    for (b, g, be, Wn) in layer_params:
        accp = edge_k(hwp, src3d, dst3d, zrow)
        b_2d, g_2d, be_2d = b.reshape(1, D), g.reshape(1, D), be.reshape(1, D)
        if Wn is None:
            h = pl.pallas_call(
                _t3_body,
                grid=grid,
                in_specs=[_row_spec(D), _row_spec(D), accp_spec, dis_spec,
                          full(1, D), full(1, D), full(1, D)],
                out_specs=_row_spec(D),
                out_shape=jax.ShapeDtypeStruct((N, D), jnp.float32),
            )(h, hwp, accp, dis, b_2d, g_2d, be_2d)
        else:
            h, hwp = pl.pallas_call(
                _t2_body,
                grid=grid,
                in_specs=[_row_spec(D), _row_spec(D), accp_spec, dis_spec,
                          full(1, D), full(1, D), full(1, D), full(D, D)],
                out_specs=[_row_spec(D), _row_spec(D)],
                out_shape=[jax.ShapeDtypeStruct((N, D), jnp.float32),
                           jax.ShapeDtypeStruct((N, D), jnp.float32)],
            )(h, hwp, accp, dis, b_2d, g_2d, be_2d, Wn)
    return h
